# baseline (speedup 1.0000x reference)
import jax
import jax.numpy as jnp
from jax import lax
from jax.experimental import pallas as pl
from jax.experimental.pallas import tpu as pltpu
from jax.experimental.pallas import tpu_sc as plsc

KE = 14.399645351950548
CUTON = 2.5
CUTOFF = 7.5
LR_CUTOFF = 10.0
KEHALF = KE / 2
CUTON16 = CUTON ** 16
CUT_RCONSTANT = LR_CUTOFF ** 15 / (LR_CUTOFF ** 16 + CUTON ** 16) ** (17.0 / 16.0)
CUT_CONSTANT = (1.0 / (CUTON ** 16 + LR_CUTOFF ** 16) ** (1.0 / 16.0)
                + LR_CUTOFF ** 16 / (LR_CUTOFF ** 16 + CUTON ** 16) ** (17.0 / 16.0))

N = 100000
N_PAD = 100352
E = 3200000
M = 1024
NC = 2
NS = 16
NR = 8
NSLOT = 2
NW = NC * NS
EPT = E // NW
C = 2000
NCHUNK = EPT // C
GROUPS = C // 16
AC = 2048
AGROUPS = AC // 16
ACHUNKS = N_PAD // AC
APASS = -(-ACHUNKS // NS)
QCHUNK = N_PAD // NS

_L = 16

_LOG2C0 = -2.4967665314454458
_LOG2C1 = 4.028355224149915
_LOG2C2 = -2.0810447757974697
_LOG2C3 = 0.6288099265675859
_LOG2C4 = -0.0791495757779922
_NEG_LN2_16 = -0.043321698784993864


def _rsqrt(x, iters):
    bits = lax.bitcast_convert_type(x, jnp.int32)
    r = lax.bitcast_convert_type(jnp.int32(0x5F3759DF) - (bits >> 1), jnp.float32)
    for _ in range(iters):
        r = r * (1.5 - 0.5 * x * r * r)
    return r


def _edge_energy(d2, qi, qj):
    r = _rsqrt(d2, 2)
    d = d2 * r
    fac = KEHALF * qi * qj
    t = jnp.minimum(jnp.maximum((d - CUTON) * (1.0 / (CUTOFF - CUTON)), 0.0), 1.0)
    t2 = t * t
    t3 = t2 * t
    f = 1.0 - t3 * (10.0 - 15.0 * t + 6.0 * t2)
    x_c = r + d * (1.0 / (LR_CUTOFF * LR_CUTOFF)) - (2.0 / LR_CUTOFF)
    coulomb = jnp.where(d < LR_CUTOFF, x_c, jnp.zeros_like(d))
    d4 = d2 * d2
    d8 = d4 * d4
    d16 = d8 * d8
    tt = d16 + CUTON16
    tb = lax.bitcast_convert_type(tt, jnp.int32)
    ex = (tb >> 23) - 127
    mant = lax.bitcast_convert_type(
        (tb & 0x007FFFFF) | 0x3F800000, jnp.float32)
    p = _LOG2C4
    p = p * mant + _LOG2C3
    p = p * mant + _LOG2C2
    p = p * mant + _LOG2C1
    p = p * mant + _LOG2C0
    log2tt = ex.astype(jnp.float32) + p
    w = jnp.exp(log2tt * _NEG_LN2_16)
    one_m_f = 1.0 - f
    damped = w + one_m_f * CUT_RCONSTANT * d - CUT_CONSTANT
    return fac * (f * damped + one_m_f * coulomb)


def _body(q_hbm, rx_hbm, ry_hbm, rz_hbm, ii_hbm, jj_hbm, im_hbm, out_hbm,
          zb_v, ii0_v, ii1_v, jj0_v, jj1_v, rx0_v, rx1_v, ry0_v, ry1_v,
          rz0_v, rz1_v, qi0_v, qi1_v, qj0_v, qj1_v, ev_v, yv_v, im_v,
          t2_v, mb_v, q_sh, y_sh, m_sh, isem0, isem1, gsem0, gsem1):
    c = lax.axis_index("c")
    s = lax.axis_index("s")
    wid = c * NS + s
    row = s & (NR - 1)
    slot = s >> 3

    pltpu.sync_copy(q_hbm.at[pl.ds(s * QCHUNK, QCHUNK)],
                    q_sh.at[pl.ds(s * QCHUNK, QCHUNK)])

    def _zfill(i, carry):
        zb_v[pl.ds(i * _L, _L)] = jnp.zeros((_L,), jnp.float32)
        return carry
    lax.fori_loop(0, AC // _L, _zfill, 0)

    @pl.when(s < NR)
    def _():
        def _zrow(k, carry):
            pltpu.sync_copy(zb_v, y_sh.at[pl.ds(s * N_PAD + k * AC, AC)])
            return carry
        lax.fori_loop(0, ACHUNKS, _zrow, 0)

    pltpu.sync_copy(zb_v.at[pl.ds(0, M)], m_sh.at[pl.ds(s * M, M)])

    plsc.subcore_barrier()

    ebase = wid * EPT
    y_off = row * N_PAD

    ii_b = (ii0_v, ii1_v)
    jj_b = (jj0_v, jj1_v)
    rx_b = (rx0_v, rx1_v)
    ry_b = (ry0_v, ry1_v)
    rz_b = (rz0_v, rz1_v)
    qi_b = (qi0_v, qi1_v)
    qj_b = (qj0_v, qj1_v)
    isem = (isem0, isem1)
    gsem = (gsem0, gsem1)

    def _issue_idx(ck, bi):
        off = ebase + ck * C
        pltpu.async_copy(ii_hbm.at[pl.ds(off, C)], ii_b[bi], isem[bi])
        pltpu.async_copy(jj_hbm.at[pl.ds(off, C)], jj_b[bi], isem[bi])
        pltpu.async_copy(rx_hbm.at[pl.ds(off, C)], rx_b[bi], isem[bi])
        pltpu.async_copy(ry_hbm.at[pl.ds(off, C)], ry_b[bi], isem[bi])
        pltpu.async_copy(rz_hbm.at[pl.ds(off, C)], rz_b[bi], isem[bi])

    def _wait_idx(ck, bi):
        off = ebase + ck * C
        for src, dst in ((ii_hbm, ii_b[bi]), (jj_hbm, jj_b[bi]),
                         (rx_hbm, rx_b[bi]), (ry_hbm, ry_b[bi]),
                         (rz_hbm, rz_b[bi])):
            pltpu.make_async_copy(src.at[pl.ds(off, C)], dst, isem[bi]).wait()

    def _issue_gath(bi):
        pltpu.async_copy(q_sh.at[ii_b[bi]], qi_b[bi], gsem[bi])
        pltpu.async_copy(q_sh.at[jj_b[bi]], qj_b[bi], gsem[bi])

    def _wait_gath(bi):
        pltpu.make_async_copy(q_sh.at[ii_b[bi]], qi_b[bi], gsem[bi]).wait()
        pltpu.make_async_copy(q_sh.at[jj_b[bi]], qj_b[bi], gsem[bi]).wait()

    def _compute(bi):
        qi_v, qj_v = qi_b[bi], qj_b[bi]
        rx_v, ry_v, rz_v = rx_b[bi], ry_b[bi], rz_b[bi]

        @plsc.parallel_loop(0, GROUPS, unroll=25)
        def _grp(g):
            sl = pl.ds(g * _L, _L)
            qi = qi_v[sl]
            qj = qj_v[sl]
            x = rx_v[sl]
            y = ry_v[sl]
            z = rz_v[sl]
            d2 = x * x + y * y + z * z
            ev_v[sl] = _edge_energy(d2, qi, qj)

    def _scatter(bi):
        for sl in range(NSLOT):
            plsc.subcore_barrier()

            @pl.when(slot == sl)
            def _():
                pltpu.sync_copy(
                    ev_v, y_sh.at[pl.ds(y_off, N_PAD)].at[ii_b[bi]], add=True)

    _issue_idx(0, 0)
    _wait_idx(0, 0)
    _issue_gath(0)
    _issue_idx(1, 1)

    def _pair(m, carry):
        ck = m * 2
        _wait_idx(ck + 1, 1)
        _issue_gath(1)
        _wait_gath(0)
        _compute(0)
        _scatter(0)

        @pl.when(ck + 2 < NCHUNK)
        def _():
            _issue_idx(ck + 2, 0)
        _wait_gath(1)
        _compute(1)

        @pl.when(ck + 2 < NCHUNK)
        def _():
            _wait_idx(ck + 2, 0)
            _issue_gath(0)
        _scatter(1)

        @pl.when(ck + 3 < NCHUNK)
        def _():
            _issue_idx(ck + 3, 1)
        return carry
    lax.fori_loop(0, NCHUNK // 2, _pair, 0)

    plsc.subcore_barrier()

    m_off = s * M
    for a0 in range(APASS):
        ak = a0 * NS + s

        @pl.when(ak < ACHUNKS)
        def _():
            aoff = ak * AC
            pltpu.sync_copy(y_sh.at[pl.ds(aoff, AC)], yv_v)
            for t in range(1, NR):
                pltpu.sync_copy(y_sh.at[pl.ds(t * N_PAD + aoff, AC)], t2_v)

                def _acc(g, carry):
                    sl = pl.ds(g * _L, _L)
                    yv_v[sl] = yv_v[sl] + t2_v[sl]
                    return carry
                lax.fori_loop(0, AGROUPS, _acc, 0)
            pltpu.sync_copy(im_hbm.at[pl.ds(aoff, AC)], im_v)

            def _madj(g, carry):
                sl = pl.ds(g * _L, _L)
                im_v[sl] = im_v[sl] + m_off
                return carry
            lax.fori_loop(0, AGROUPS, _madj, 0)
            pltpu.sync_copy(yv_v, m_sh.at[im_v], add=True)

    plsc.subcore_barrier()

    @pl.when(s == 0)
    def _():
        pltpu.sync_copy(m_sh.at[pl.ds(0, M)], mb_v)
        for t in range(1, NS):
            pltpu.sync_copy(m_sh.at[pl.ds(t * M, M)], t2_v.at[pl.ds(0, M)])

            def _macc(g, carry):
                sl = pl.ds(g * _L, _L)
                mb_v[sl] = mb_v[sl] + t2_v[sl]
                return carry
            lax.fori_loop(0, M // _L, _macc, 0)
        pltpu.sync_copy(mb_v, out_hbm.at[c])


_sc_call = pl.kernel(
    _body,
    out_type=jax.ShapeDtypeStruct((NC, M), jnp.float32),
    mesh=plsc.VectorSubcoreMesh(core_axis_name="c", subcore_axis_name="s"),
    compiler_params=pltpu.CompilerParams(needs_layout_passes=False),
    scratch_types=[
        pltpu.VMEM((AC,), jnp.float32),
        pltpu.VMEM((C,), jnp.int32),
        pltpu.VMEM((C,), jnp.int32),
        pltpu.VMEM((C,), jnp.int32),
        pltpu.VMEM((C,), jnp.int32),
        pltpu.VMEM((C,), jnp.float32),
        pltpu.VMEM((C,), jnp.float32),
        pltpu.VMEM((C,), jnp.float32),
        pltpu.VMEM((C,), jnp.float32),
        pltpu.VMEM((C,), jnp.float32),
        pltpu.VMEM((C,), jnp.float32),
        pltpu.VMEM((C,), jnp.float32),
        pltpu.VMEM((C,), jnp.float32),
        pltpu.VMEM((C,), jnp.float32),
        pltpu.VMEM((C,), jnp.float32),
        pltpu.VMEM((C,), jnp.float32),
        pltpu.VMEM((AC,), jnp.float32),
        pltpu.VMEM((AC,), jnp.int32),
        pltpu.VMEM((AC,), jnp.float32),
        pltpu.VMEM((M,), jnp.float32),
        pltpu.VMEM_SHARED((N_PAD,), jnp.float32),
        pltpu.VMEM_SHARED((NR * N_PAD,), jnp.float32),
        pltpu.VMEM_SHARED((NS * M,), jnp.float32),
        pltpu.SemaphoreType.DMA,
        pltpu.SemaphoreType.DMA,
        pltpu.SemaphoreType.DMA,
        pltpu.SemaphoreType.DMA,
    ],
)


def kernel(Z, partial_charges, Rij, idx_i, idx_j, idx_m):
    q = jnp.squeeze(partial_charges, -1)
    q_pad = jnp.concatenate([q, jnp.zeros((N_PAD - N,), jnp.float32)])
    im_pad = jnp.concatenate(
        [idx_m.astype(jnp.int32), jnp.zeros((N_PAD - N,), jnp.int32)])
    out2 = _sc_call(q_pad, Rij[:, 0], Rij[:, 1], Rij[:, 2],
                    idx_i.astype(jnp.int32), idx_j.astype(jnp.int32),
                    im_pad)
    return out2[0] + out2[1]


# speedup vs baseline: 1.0325x; 1.0325x over previous
import jax
import jax.numpy as jnp
from jax import lax
from jax.experimental import pallas as pl
from jax.experimental.pallas import tpu as pltpu
from jax.experimental.pallas import tpu_sc as plsc

KE = 14.399645351950548
CUTON = 2.5
CUTOFF = 7.5
LR_CUTOFF = 10.0
KEHALF = KE / 2
CUTON16 = CUTON ** 16
CUT_RCONSTANT = LR_CUTOFF ** 15 / (LR_CUTOFF ** 16 + CUTON ** 16) ** (17.0 / 16.0)
CUT_CONSTANT = (1.0 / (CUTON ** 16 + LR_CUTOFF ** 16) ** (1.0 / 16.0)
                + LR_CUTOFF ** 16 / (LR_CUTOFF ** 16 + CUTON ** 16) ** (17.0 / 16.0))

N = 100000
N_PAD = 100352
E = 3200000
M = 1024
NC = 2
NS = 16
NR = 8
NSLOT = 2
NW = NC * NS
EPT = E // NW
C = 4000
NCHUNK = EPT // C
GROUPS = C // 16
AC = 2048
AGROUPS = AC // 16
ACHUNKS = N_PAD // AC
APASS = -(-ACHUNKS // NS)
QCHUNK = N_PAD // NS

_L = 16

_LOG2C0 = -2.4967665314454458
_LOG2C1 = 4.028355224149915
_LOG2C2 = -2.0810447757974697
_LOG2C3 = 0.6288099265675859
_LOG2C4 = -0.0791495757779922
_NEG_LN2_16 = -0.043321698784993864


def _rsqrt(x, iters):
    bits = lax.bitcast_convert_type(x, jnp.int32)
    r = lax.bitcast_convert_type(jnp.int32(0x5F3759DF) - (bits >> 1), jnp.float32)
    for _ in range(iters):
        r = r * (1.5 - 0.5 * x * r * r)
    return r


def _edge_energy(d2, qi, qj):
    r = _rsqrt(d2, 2)
    d = d2 * r
    fac = KEHALF * qi * qj
    t = jnp.minimum(jnp.maximum((d - CUTON) * (1.0 / (CUTOFF - CUTON)), 0.0), 1.0)
    t2 = t * t
    t3 = t2 * t
    f = 1.0 - t3 * (10.0 - 15.0 * t + 6.0 * t2)
    x_c = r + d * (1.0 / (LR_CUTOFF * LR_CUTOFF)) - (2.0 / LR_CUTOFF)
    coulomb = jnp.where(d < LR_CUTOFF, x_c, jnp.zeros_like(d))
    d4 = d2 * d2
    d8 = d4 * d4
    d16 = d8 * d8
    tt = d16 + CUTON16
    tb = lax.bitcast_convert_type(tt, jnp.int32)
    ex = (tb >> 23) - 127
    mant = lax.bitcast_convert_type(
        (tb & 0x007FFFFF) | 0x3F800000, jnp.float32)
    p = _LOG2C4
    p = p * mant + _LOG2C3
    p = p * mant + _LOG2C2
    p = p * mant + _LOG2C1
    p = p * mant + _LOG2C0
    log2tt = ex.astype(jnp.float32) + p
    w = jnp.exp(log2tt * _NEG_LN2_16)
    one_m_f = 1.0 - f
    damped = w + one_m_f * CUT_RCONSTANT * d - CUT_CONSTANT
    return fac * (f * damped + one_m_f * coulomb)


def _body(q_hbm, rx_hbm, ry_hbm, rz_hbm, ii_hbm, jj_hbm, im_hbm, out_hbm,
          zb_v, ii0_v, ii1_v, jj0_v, jj1_v, rx0_v, rx1_v, ry0_v, ry1_v,
          rz0_v, rz1_v, qi0_v, qi1_v, qj0_v, qj1_v, ev_v, yv_v, im_v,
          t2_v, mb_v, q_sh, y_sh, m_sh, isem0, isem1, gsem0, gsem1):
    c = lax.axis_index("c")
    s = lax.axis_index("s")
    wid = c * NS + s
    row = s & (NR - 1)
    slot = s >> 3

    pltpu.sync_copy(q_hbm.at[pl.ds(s * QCHUNK, QCHUNK)],
                    q_sh.at[pl.ds(s * QCHUNK, QCHUNK)])

    def _zfill(i, carry):
        zb_v[pl.ds(i * _L, _L)] = jnp.zeros((_L,), jnp.float32)
        return carry
    lax.fori_loop(0, AC // _L, _zfill, 0)

    @pl.when(s < NR)
    def _():
        def _zrow(k, carry):
            pltpu.sync_copy(zb_v, y_sh.at[pl.ds(s * N_PAD + k * AC, AC)])
            return carry
        lax.fori_loop(0, ACHUNKS, _zrow, 0)

    pltpu.sync_copy(zb_v.at[pl.ds(0, M)], m_sh.at[pl.ds(s * M, M)])

    plsc.subcore_barrier()

    ebase = wid * EPT
    y_off = row * N_PAD

    ii_b = (ii0_v, ii1_v)
    jj_b = (jj0_v, jj1_v)
    rx_b = (rx0_v, rx1_v)
    ry_b = (ry0_v, ry1_v)
    rz_b = (rz0_v, rz1_v)
    qi_b = (qi0_v, qi1_v)
    qj_b = (qj0_v, qj1_v)
    isem = (isem0, isem1)
    gsem = (gsem0, gsem1)

    def _issue_idx(ck, bi):
        off = ebase + ck * C
        pltpu.async_copy(ii_hbm.at[pl.ds(off, C)], ii_b[bi], isem[bi])
        pltpu.async_copy(jj_hbm.at[pl.ds(off, C)], jj_b[bi], isem[bi])
        pltpu.async_copy(rx_hbm.at[pl.ds(off, C)], rx_b[bi], isem[bi])
        pltpu.async_copy(ry_hbm.at[pl.ds(off, C)], ry_b[bi], isem[bi])
        pltpu.async_copy(rz_hbm.at[pl.ds(off, C)], rz_b[bi], isem[bi])

    def _wait_idx(ck, bi):
        off = ebase + ck * C
        for src, dst in ((ii_hbm, ii_b[bi]), (jj_hbm, jj_b[bi]),
                         (rx_hbm, rx_b[bi]), (ry_hbm, ry_b[bi]),
                         (rz_hbm, rz_b[bi])):
            pltpu.make_async_copy(src.at[pl.ds(off, C)], dst, isem[bi]).wait()

    def _issue_gath(bi):
        pltpu.async_copy(q_sh.at[ii_b[bi]], qi_b[bi], gsem[bi])
        pltpu.async_copy(q_sh.at[jj_b[bi]], qj_b[bi], gsem[bi])

    def _wait_gath(bi):
        pltpu.make_async_copy(q_sh.at[ii_b[bi]], qi_b[bi], gsem[bi]).wait()
        pltpu.make_async_copy(q_sh.at[jj_b[bi]], qj_b[bi], gsem[bi]).wait()

    def _compute(bi):
        qi_v, qj_v = qi_b[bi], qj_b[bi]
        rx_v, ry_v, rz_v = rx_b[bi], ry_b[bi], rz_b[bi]

        @plsc.parallel_loop(0, GROUPS, unroll=5)
        def _grp(g):
            sl = pl.ds(g * _L, _L)
            qi = qi_v[sl]
            qj = qj_v[sl]
            x = rx_v[sl]
            y = ry_v[sl]
            z = rz_v[sl]
            d2 = x * x + y * y + z * z
            ev_v[sl] = _edge_energy(d2, qi, qj)

    def _scatter(bi):
        for sl in range(NSLOT):
            plsc.subcore_barrier()

            @pl.when(slot == sl)
            def _():
                pltpu.sync_copy(
                    ev_v, y_sh.at[pl.ds(y_off, N_PAD)].at[ii_b[bi]], add=True)

    _issue_idx(0, 0)
    _wait_idx(0, 0)
    _issue_gath(0)
    _issue_idx(1, 1)

    def _pair(m, carry):
        ck = m * 2
        _wait_idx(ck + 1, 1)
        _issue_gath(1)
        _wait_gath(0)
        _compute(0)
        _scatter(0)

        @pl.when(ck + 2 < NCHUNK)
        def _():
            _issue_idx(ck + 2, 0)
        _wait_gath(1)
        _compute(1)

        @pl.when(ck + 2 < NCHUNK)
        def _():
            _wait_idx(ck + 2, 0)
            _issue_gath(0)
        _scatter(1)

        @pl.when(ck + 3 < NCHUNK)
        def _():
            _issue_idx(ck + 3, 1)
        return carry
    lax.fori_loop(0, NCHUNK // 2, _pair, 0)

    plsc.subcore_barrier()

    m_off = s * M
    for a0 in range(APASS):
        ak = a0 * NS + s

        @pl.when(ak < ACHUNKS)
        def _():
            aoff = ak * AC
            pltpu.sync_copy(y_sh.at[pl.ds(aoff, AC)], yv_v)
            for t in range(1, NR):
                pltpu.sync_copy(y_sh.at[pl.ds(t * N_PAD + aoff, AC)], t2_v)

                def _acc(g, carry):
                    sl = pl.ds(g * _L, _L)
                    yv_v[sl] = yv_v[sl] + t2_v[sl]
                    return carry
                lax.fori_loop(0, AGROUPS, _acc, 0)
            pltpu.sync_copy(im_hbm.at[pl.ds(aoff, AC)], im_v)

            def _madj(g, carry):
                sl = pl.ds(g * _L, _L)
                im_v[sl] = im_v[sl] + m_off
                return carry
            lax.fori_loop(0, AGROUPS, _madj, 0)
            pltpu.sync_copy(yv_v, m_sh.at[im_v], add=True)

    plsc.subcore_barrier()

    @pl.when(s == 0)
    def _():
        pltpu.sync_copy(m_sh.at[pl.ds(0, M)], mb_v)
        for t in range(1, NS):
            pltpu.sync_copy(m_sh.at[pl.ds(t * M, M)], t2_v.at[pl.ds(0, M)])

            def _macc(g, carry):
                sl = pl.ds(g * _L, _L)
                mb_v[sl] = mb_v[sl] + t2_v[sl]
                return carry
            lax.fori_loop(0, M // _L, _macc, 0)
        pltpu.sync_copy(mb_v, out_hbm.at[c])


_sc_call = pl.kernel(
    _body,
    out_type=jax.ShapeDtypeStruct((NC, M), jnp.float32),
    mesh=plsc.VectorSubcoreMesh(core_axis_name="c", subcore_axis_name="s"),
    compiler_params=pltpu.CompilerParams(needs_layout_passes=False),
    scratch_types=[
        pltpu.VMEM((AC,), jnp.float32),
        pltpu.VMEM((C,), jnp.int32),
        pltpu.VMEM((C,), jnp.int32),
        pltpu.VMEM((C,), jnp.int32),
        pltpu.VMEM((C,), jnp.int32),
        pltpu.VMEM((C,), jnp.float32),
        pltpu.VMEM((C,), jnp.float32),
        pltpu.VMEM((C,), jnp.float32),
        pltpu.VMEM((C,), jnp.float32),
        pltpu.VMEM((C,), jnp.float32),
        pltpu.VMEM((C,), jnp.float32),
        pltpu.VMEM((C,), jnp.float32),
        pltpu.VMEM((C,), jnp.float32),
        pltpu.VMEM((C,), jnp.float32),
        pltpu.VMEM((C,), jnp.float32),
        pltpu.VMEM((C,), jnp.float32),
        pltpu.VMEM((AC,), jnp.float32),
        pltpu.VMEM((AC,), jnp.int32),
        pltpu.VMEM((AC,), jnp.float32),
        pltpu.VMEM((M,), jnp.float32),
        pltpu.VMEM_SHARED((N_PAD,), jnp.float32),
        pltpu.VMEM_SHARED((NR * N_PAD,), jnp.float32),
        pltpu.VMEM_SHARED((NS * M,), jnp.float32),
        pltpu.SemaphoreType.DMA,
        pltpu.SemaphoreType.DMA,
        pltpu.SemaphoreType.DMA,
        pltpu.SemaphoreType.DMA,
    ],
)


def kernel(Z, partial_charges, Rij, idx_i, idx_j, idx_m):
    q = jnp.squeeze(partial_charges, -1)
    q_pad = jnp.concatenate([q, jnp.zeros((N_PAD - N,), jnp.float32)])
    im_pad = jnp.concatenate(
        [idx_m.astype(jnp.int32), jnp.zeros((N_PAD - N,), jnp.int32)])
    out2 = _sc_call(q_pad, Rij[:, 0], Rij[:, 1], Rij[:, 2],
                    idx_i.astype(jnp.int32), idx_j.astype(jnp.int32),
                    im_pad)
    return out2[0] + out2[1]
